# baseline (speedup 1.0000x reference)
import functools

import jax
import jax.numpy as jnp
import numpy as np
from jax import lax
from jax.experimental import pallas as pl
from jax.experimental.pallas import tpu as pltpu
from jax.experimental.pallas import tpu_sc as plsc

_BATCH = 16384
_EMBED_DIM = 64
_OUT_FEATURES = 64
_ROW_W = 128

_NC = 2
_NS = 16
_NW = _NC * _NS
_B_PER_W = _BATCH // _NW


def _sc_gather(table2, idx2):
    mesh = plsc.VectorSubcoreMesh(core_axis_name="c", subcore_axis_name="s")

    @functools.partial(
        pl.kernel,
        mesh=mesh,
        out_type=jax.ShapeDtypeStruct((_BATCH, _ROW_W), jnp.float32),
        scratch_types=[
            pltpu.VMEM((_B_PER_W,), jnp.int32),
            pltpu.VMEM((_B_PER_W, _ROW_W), jnp.float32),
            pltpu.SemaphoreType.DMA,
        ],
    )
    def gather_kernel(table_hbm, idx_hbm, out_hbm, idx_v, rows_v, sem):
        wid = lax.axis_index("s") * _NC + lax.axis_index("c")
        base = wid * _B_PER_W
        pltpu.sync_copy(idx_hbm.at[pl.ds(base, _B_PER_W)], idx_v)
        pltpu.async_copy(table_hbm.at[idx_v], rows_v, sem).wait()
        pltpu.sync_copy(rows_v, out_hbm.at[pl.ds(base, _B_PER_W)])

    return gather_kernel(table2, idx2)


def _tc_dense(x2, parity, dense_kernel, dense_bias):
    he_const = 1.0 / np.sqrt(dense_kernel.shape[0])
    bias2d = dense_bias.reshape(1, _OUT_FEATURES)
    blk = 2048

    def dense_kernel_body(x_ref, p_ref, w_ref, b_ref, o_ref):
        lo = x_ref[:, :_EMBED_DIM]
        hi = x_ref[:, _EMBED_DIM:]
        x = jnp.where(p_ref[...] == 1, hi, lo)
        acc = jnp.dot(x, w_ref[...], preferred_element_type=jnp.float32)
        o_ref[...] = acc * he_const + b_ref[...]

    return pl.pallas_call(
        dense_kernel_body,
        grid=(_BATCH // blk,),
        in_specs=[
            pl.BlockSpec((blk, _ROW_W), lambda i: (i, 0)),
            pl.BlockSpec((blk, 1), lambda i: (i, 0)),
            pl.BlockSpec((_EMBED_DIM, _OUT_FEATURES), lambda i: (0, 0)),
            pl.BlockSpec((1, _OUT_FEATURES), lambda i: (0, 0)),
        ],
        out_specs=pl.BlockSpec((blk, _OUT_FEATURES), lambda i: (i, 0)),
        out_shape=jax.ShapeDtypeStruct((_BATCH, _OUT_FEATURES), jnp.float32),
    )(x2, parity, dense_kernel, bias2d)


def kernel(label, embed_table, dense_kernel, dense_bias):
    table2 = embed_table.reshape(-1, _ROW_W)
    idx2 = label // 2
    parity = (label % 2).astype(jnp.int32).reshape(_BATCH, 1)
    x2 = _sc_gather(table2, idx2)
    y = _tc_dense(x2, parity, dense_kernel, dense_bias)
    return y.reshape(-1, 8, 8, 1)


# speedup vs baseline: 1.0277x; 1.0277x over previous
import functools

import jax
import jax.numpy as jnp
import numpy as np
from jax import lax
from jax.experimental import pallas as pl
from jax.experimental.pallas import tpu as pltpu
from jax.experimental.pallas import tpu_sc as plsc

_BATCH = 16384
_EMBED_DIM = 64
_OUT_FEATURES = 64

_NC = 2
_NS = 16
_NW = _NC * _NS
_B_PER_W = _BATCH // _NW


def _sc_gather(embed_table, label):
    mesh = plsc.VectorSubcoreMesh(core_axis_name="c", subcore_axis_name="s")

    @functools.partial(
        pl.kernel,
        mesh=mesh,
        out_type=jax.ShapeDtypeStruct((_BATCH, _EMBED_DIM), jnp.float32),
        scratch_types=[
            pltpu.VMEM((_B_PER_W,), jnp.int32),
            pltpu.SemaphoreType.DMA,
        ],
    )
    def gather_kernel(table_hbm, idx_hbm, out_hbm, idx_v, sem):
        wid = lax.axis_index("s") * _NC + lax.axis_index("c")
        base = wid * _B_PER_W
        pltpu.sync_copy(idx_hbm.at[pl.ds(base, _B_PER_W)], idx_v)

        @pl.loop(0, _B_PER_W, step=16)
        def _(i):
            chunk = idx_v[pl.ds(i, 16)]
            for j in range(16):
                pltpu.make_async_copy(
                    table_hbm.at[chunk[j]], out_hbm.at[base + i + j], sem
                ).start()

        pltpu.make_async_copy(
            table_hbm.at[pl.ds(0, _B_PER_W)],
            out_hbm.at[pl.ds(base, _B_PER_W)],
            sem,
        ).wait()

    return gather_kernel(embed_table, label)


def _tc_dense(x, dense_kernel, dense_bias):
    he_const = 1.0 / np.sqrt(dense_kernel.shape[0])
    bias2d = dense_bias.reshape(1, _OUT_FEATURES)
    blk = 2048

    def dense_kernel_body(x_ref, w_ref, b_ref, o_ref):
        acc = jnp.dot(x_ref[...], w_ref[...], preferred_element_type=jnp.float32)
        o_ref[...] = acc * he_const + b_ref[...]

    return pl.pallas_call(
        dense_kernel_body,
        grid=(_BATCH // blk,),
        in_specs=[
            pl.BlockSpec((blk, _EMBED_DIM), lambda i: (i, 0)),
            pl.BlockSpec((_EMBED_DIM, _OUT_FEATURES), lambda i: (0, 0)),
            pl.BlockSpec((1, _OUT_FEATURES), lambda i: (0, 0)),
        ],
        out_specs=pl.BlockSpec((blk, _OUT_FEATURES), lambda i: (i, 0)),
        out_shape=jax.ShapeDtypeStruct((_BATCH, _OUT_FEATURES), jnp.float32),
    )(x, dense_kernel, bias2d)


def kernel(label, embed_table, dense_kernel, dense_bias):
    x = _sc_gather(embed_table, label)
    y = _tc_dense(x, dense_kernel, dense_bias)
    return y.reshape(-1, 8, 8, 1)


# speedup vs baseline: 3.0329x; 2.9511x over previous
import functools

import jax
import jax.numpy as jnp
import numpy as np
from jax import lax
from jax.experimental import pallas as pl
from jax.experimental.pallas import tpu as pltpu
from jax.experimental.pallas import tpu_sc as plsc

_BATCH = 16384
_EMBED_DIM = 64
_OUT_FEATURES = 64
_NUM_CLASSES = 1000000
_PACK = 524288
_P2_ROWS = _PACK // 2
_ROW_W = 128
_PROJ_BLK = 8192

_NC = 2
_NS = 16
_NW = _NC * _NS
_B_PER_W = _BATCH // _NW


def _tc_project(table_t, dense_kernel):
    he_const = 1.0 / np.sqrt(dense_kernel.shape[0])
    n_steps = _PACK // _PROJ_BLK
    last_blk = (_NUM_CLASSES - 1) // _PROJ_BLK

    def proj_body(xlo_ref, xhi_ref, w_ref, o_ref):
        wb = (w_ref[...] * he_const).astype(jnp.bfloat16)
        ys = []
        for x_ref in (xlo_ref, xhi_ref):
            xb = x_ref[...].astype(jnp.bfloat16)
            y = jax.lax.dot_general(
                xb, wb, (((0,), (0,)), ((), ())),
                preferred_element_type=jnp.float32,
            )
            ys.append(y.astype(jnp.bfloat16))
        packed = jnp.concatenate(ys, axis=1)
        o_ref[...] = pltpu.bitcast(packed, jnp.float32)

    qb = n_steps
    return pl.pallas_call(
        proj_body,
        grid=(n_steps,),
        in_specs=[
            pl.BlockSpec((_EMBED_DIM, _PROJ_BLK), lambda i: (0, i)),
            pl.BlockSpec(
                (_EMBED_DIM, _PROJ_BLK),
                lambda i: (0, jnp.minimum(i + qb, last_blk)),
            ),
            pl.BlockSpec((_EMBED_DIM, _OUT_FEATURES), lambda i: (0, 0)),
        ],
        out_specs=pl.BlockSpec((_PROJ_BLK // 2, _ROW_W), lambda i: (i, 0)),
        out_shape=jax.ShapeDtypeStruct((_P2_ROWS, _ROW_W), jnp.float32),
    )(table_t, table_t, dense_kernel)


def _sc_gather(p2, idx):
    mesh = plsc.VectorSubcoreMesh(core_axis_name="c", subcore_axis_name="s")

    @functools.partial(
        pl.kernel,
        mesh=mesh,
        out_type=jax.ShapeDtypeStruct((_BATCH, _ROW_W), jnp.float32),
        scratch_types=[
            pltpu.VMEM((_B_PER_W,), jnp.int32),
            pltpu.VMEM((_B_PER_W, _ROW_W), jnp.float32),
            pltpu.SemaphoreType.DMA,
        ],
    )
    def gather_kernel(table_hbm, idx_hbm, out_hbm, idx_v, rows_v, sem):
        wid = lax.axis_index("s") * _NC + lax.axis_index("c")
        base = wid * _B_PER_W
        pltpu.sync_copy(idx_hbm.at[pl.ds(base, _B_PER_W)], idx_v)
        pltpu.async_copy(table_hbm.at[idx_v], rows_v, sem).wait()
        pltpu.sync_copy(rows_v, out_hbm.at[pl.ds(base, _B_PER_W)])

    return gather_kernel(p2, idx)


def _tc_epilogue(g, subrow, half, dense_bias):
    bias2d = dense_bias.reshape(1, _OUT_FEATURES)
    blk = 4096

    def epi_body(g_ref, s_ref, h_ref, b_ref, o_ref):
        gi = pltpu.bitcast(g_ref[...], jnp.int32)
        lo_bits = gi << 16
        hi_bits = gi & jnp.int32(-65536)
        sel = jnp.where(s_ref[...] == 0, lo_bits, hi_bits)
        yf = pltpu.bitcast(sel, jnp.float32)
        d = _OUT_FEATURES
        y = jnp.where(h_ref[...] == 0, yf[:, :d], yf[:, d:])
        o_ref[...] = y + b_ref[...]

    return pl.pallas_call(
        epi_body,
        grid=(_BATCH // blk,),
        in_specs=[
            pl.BlockSpec((blk, _ROW_W), lambda i: (i, 0)),
            pl.BlockSpec((blk, 1), lambda i: (i, 0)),
            pl.BlockSpec((blk, 1), lambda i: (i, 0)),
            pl.BlockSpec((1, _OUT_FEATURES), lambda i: (0, 0)),
        ],
        out_specs=pl.BlockSpec((blk, _OUT_FEATURES), lambda i: (i, 0)),
        out_shape=jax.ShapeDtypeStruct((_BATCH, _OUT_FEATURES), jnp.float32),
    )(g, subrow, half, bias2d)


def kernel(label, embed_table, dense_kernel, dense_bias):
    table_t = embed_table.T
    p2 = _tc_project(table_t, dense_kernel)
    lo = label % _PACK
    idx = lo // 2
    subrow = (lo % 2).astype(jnp.int32).reshape(_BATCH, 1)
    half = (label // _PACK).astype(jnp.int32).reshape(_BATCH, 1)
    g = _sc_gather(p2, idx)
    y = _tc_epilogue(g, subrow, half, dense_bias)
    return y.reshape(-1, 8, 8, 1)


# speedup vs baseline: 3.4661x; 1.1428x over previous
import functools

import jax
import jax.numpy as jnp
import numpy as np
from jax import lax
from jax.experimental import pallas as pl
from jax.experimental.pallas import tpu as pltpu
from jax.experimental.pallas import tpu_sc as plsc

_BATCH = 16384
_EMBED_DIM = 64
_OUT_FEATURES = 64
_NUM_CLASSES = 1000000
_PACK = 524288
_P2_ROWS = _PACK // 2
_ROW_W = 128
_PROJ_BLK = 16384

_NC = 2
_NS = 16
_NW = _NC * _NS
_B_PER_W = _BATCH // _NW


def _tc_project(table_t, dense_kernel):
    he_const = 1.0 / np.sqrt(dense_kernel.shape[0])
    n_steps = _PACK // _PROJ_BLK
    last_blk = (_NUM_CLASSES - 1) // _PROJ_BLK

    def proj_body(xlo_ref, xhi_ref, w_ref, o_ref):
        wb = (w_ref[...] * he_const).astype(jnp.bfloat16)
        ys = []
        for x_ref in (xlo_ref, xhi_ref):
            xb = x_ref[...].astype(jnp.bfloat16)
            y = jax.lax.dot_general(
                xb, wb, (((0,), (0,)), ((), ())),
                preferred_element_type=jnp.float32,
            )
            ys.append(y.astype(jnp.bfloat16))
        packed = jnp.concatenate(ys, axis=1)
        o_ref[...] = pltpu.bitcast(packed, jnp.float32)

    qb = n_steps
    return pl.pallas_call(
        proj_body,
        grid=(n_steps,),
        in_specs=[
            pl.BlockSpec((_EMBED_DIM, _PROJ_BLK), lambda i: (0, i)),
            pl.BlockSpec(
                (_EMBED_DIM, _PROJ_BLK),
                lambda i: (0, jnp.minimum(i + qb, last_blk)),
            ),
            pl.BlockSpec((_EMBED_DIM, _OUT_FEATURES), lambda i: (0, 0)),
        ],
        out_specs=pl.BlockSpec((_PROJ_BLK // 2, _ROW_W), lambda i: (i, 0)),
        out_shape=jax.ShapeDtypeStruct((_P2_ROWS, _ROW_W), jnp.float32),
        compiler_params=pltpu.CompilerParams(
            dimension_semantics=("parallel",),
            fuse_transposed_lhs_in_matmul=True,
        ),
    )(table_t, table_t, dense_kernel)


def _sc_gather(p2, idx):
    mesh = plsc.VectorSubcoreMesh(core_axis_name="c", subcore_axis_name="s")

    @functools.partial(
        pl.kernel,
        mesh=mesh,
        out_type=jax.ShapeDtypeStruct((_BATCH, _ROW_W), jnp.float32),
        scratch_types=[
            pltpu.VMEM((_B_PER_W,), jnp.int32),
            pltpu.VMEM((_B_PER_W, _ROW_W), jnp.float32),
            pltpu.SemaphoreType.DMA,
        ],
    )
    def gather_kernel(table_hbm, idx_hbm, out_hbm, idx_v, rows_v, sem):
        wid = lax.axis_index("s") * _NC + lax.axis_index("c")
        base = wid * _B_PER_W
        pltpu.sync_copy(idx_hbm.at[pl.ds(base, _B_PER_W)], idx_v)
        pltpu.async_copy(table_hbm.at[idx_v], rows_v, sem).wait()
        pltpu.sync_copy(rows_v, out_hbm.at[pl.ds(base, _B_PER_W)])

    return gather_kernel(p2, idx)


def _tc_epilogue(g, sel, dense_bias):
    bias2d = dense_bias.reshape(1, _OUT_FEATURES)
    blk = 4096

    def epi_body(g_ref, s_ref, b_ref, o_ref):
        gi = pltpu.bitcast(g_ref[...], jnp.int32)
        s = s_ref[...].astype(jnp.int32)
        lo_bits = gi << 16
        hi_bits = gi & jnp.int32(-65536)
        picked = jnp.where((s & 1) == 0, lo_bits, hi_bits)
        yf = pltpu.bitcast(picked, jnp.float32)
        d = _OUT_FEATURES
        y = jnp.where((s & 2) == 0, yf[:, :d], yf[:, d:])
        o_ref[...] = y + b_ref[...]

    return pl.pallas_call(
        epi_body,
        grid=(_BATCH // blk,),
        in_specs=[
            pl.BlockSpec((blk, _ROW_W), lambda i: (i, 0)),
            pl.BlockSpec((blk, 1), lambda i: (i, 0)),
            pl.BlockSpec((1, _OUT_FEATURES), lambda i: (0, 0)),
        ],
        out_specs=pl.BlockSpec((blk, _OUT_FEATURES), lambda i: (i, 0)),
        out_shape=jax.ShapeDtypeStruct((_BATCH, _OUT_FEATURES), jnp.float32),
    )(g, sel, bias2d)


def kernel(label, embed_table, dense_kernel, dense_bias):
    table_t = embed_table.T
    p2 = _tc_project(table_t, dense_kernel)
    lo = label % _PACK
    idx = lo // 2
    sel = ((lo % 2) + 2 * (label // _PACK)).astype(jnp.int8).reshape(_BATCH, 1)
    g = _sc_gather(p2, idx)
    y = _tc_epilogue(g, sel, dense_bias)
    return y.reshape(-1, 8, 8, 1)


# speedup vs baseline: 3.5034x; 1.0108x over previous
import functools

import jax
import jax.numpy as jnp
import numpy as np
from jax import lax
from jax.experimental import pallas as pl
from jax.experimental.pallas import tpu as pltpu
from jax.experimental.pallas import tpu_sc as plsc

_BATCH = 16384
_EMBED_DIM = 64
_OUT_FEATURES = 64
_NUM_CLASSES = 1000000
_PACK = 524288
_P2_ROWS = _PACK // 2
_ROW_W = 128
_PROJ_BLK = 16384

_NC = 2
_NS = 16
_NW = _NC * _NS
_B_PER_W = _BATCH // _NW


def _tc_project(table_t, dense_kernel):
    he_const = 1.0 / np.sqrt(dense_kernel.shape[0])
    n_steps = _PACK // _PROJ_BLK
    last_blk = (_NUM_CLASSES - 1) // _PROJ_BLK

    def proj_body(xlo_ref, xhi_ref, w_ref, o_ref):
        wb = (w_ref[...] * he_const).astype(jnp.bfloat16)
        ys = []
        for x_ref in (xlo_ref, xhi_ref):
            xb = x_ref[...].astype(jnp.bfloat16)
            y = jax.lax.dot_general(
                xb, wb, (((0,), (0,)), ((), ())),
                preferred_element_type=jnp.float32,
            )
            ys.append(y.astype(jnp.bfloat16))
        packed = jnp.concatenate(ys, axis=1)
        o_ref[...] = pltpu.bitcast(packed, jnp.float32)

    qb = n_steps
    return pl.pallas_call(
        proj_body,
        grid=(n_steps,),
        in_specs=[
            pl.BlockSpec((_EMBED_DIM, _PROJ_BLK), lambda i: (0, i)),
            pl.BlockSpec(
                (_EMBED_DIM, _PROJ_BLK),
                lambda i: (0, jnp.minimum(i + qb, last_blk)),
            ),
            pl.BlockSpec((_EMBED_DIM, _OUT_FEATURES), lambda i: (0, 0)),
        ],
        out_specs=pl.BlockSpec((_PROJ_BLK // 2, _ROW_W), lambda i: (i, 0)),
        out_shape=jax.ShapeDtypeStruct((_P2_ROWS, _ROW_W), jnp.float32),
        compiler_params=pltpu.CompilerParams(
            dimension_semantics=("parallel",),
            fuse_transposed_lhs_in_matmul=True,
        ),
    )(table_t, table_t, dense_kernel)


def _sc_gather(p2, idx):
    mesh = plsc.VectorSubcoreMesh(core_axis_name="c", subcore_axis_name="s")

    @functools.partial(
        pl.kernel,
        mesh=mesh,
        out_type=jax.ShapeDtypeStruct((_BATCH, _ROW_W), jnp.float32),
        scratch_types=[
            pltpu.VMEM((_B_PER_W,), jnp.int32),
            pltpu.VMEM((_B_PER_W, _ROW_W), jnp.float32),
            pltpu.SemaphoreType.DMA,
        ],
    )
    def gather_kernel(table_hbm, idx_hbm, out_hbm, idx_v, rows_v, sem):
        wid = lax.axis_index("s") * _NC + lax.axis_index("c")
        base = wid * _B_PER_W
        pltpu.sync_copy(idx_hbm.at[pl.ds(base, _B_PER_W)], idx_v)
        pltpu.async_copy(table_hbm.at[idx_v], rows_v, sem).wait()
        pltpu.sync_copy(rows_v, out_hbm.at[pl.ds(base, _B_PER_W)])

    return gather_kernel(p2, idx)


def _tc_epilogue(g, sel, dense_bias):
    bias2d = dense_bias.reshape(1, _OUT_FEATURES)
    blk = 8192

    def epi_body(g_ref, s_ref, b_ref, o_ref):
        gi = pltpu.bitcast(g_ref[...], jnp.int32)
        s = s_ref[...].astype(jnp.int32)
        lo_bits = gi << 16
        hi_bits = gi & jnp.int32(-65536)
        picked = jnp.where((s & 1) == 0, lo_bits, hi_bits)
        yf = pltpu.bitcast(picked, jnp.float32)
        d = _OUT_FEATURES
        y = jnp.where((s & 2) == 0, yf[:, :d], yf[:, d:])
        o_ref[...] = (y + b_ref[...]).T

    return pl.pallas_call(
        epi_body,
        grid=(_BATCH // blk,),
        in_specs=[
            pl.BlockSpec((blk, _ROW_W), lambda i: (i, 0)),
            pl.BlockSpec((blk, 1), lambda i: (i, 0)),
            pl.BlockSpec((1, _OUT_FEATURES), lambda i: (0, 0)),
        ],
        out_specs=pl.BlockSpec((_OUT_FEATURES, blk), lambda i: (0, i)),
        out_shape=jax.ShapeDtypeStruct((_OUT_FEATURES, _BATCH), jnp.float32),
    )(g, sel, bias2d)


def kernel(label, embed_table, dense_kernel, dense_bias):
    table_t = embed_table.T
    p2 = _tc_project(table_t, dense_kernel)
    lo = label % _PACK
    idx = lo // 2
    sel = ((lo % 2) + 2 * (label // _PACK)).astype(jnp.int8).reshape(_BATCH, 1)
    g = _sc_gather(p2, idx)
    y_t = _tc_epilogue(g, sel, dense_bias)
    return y_t.T.reshape(-1, 8, 8, 1)


# speedup vs baseline: 3.6124x; 1.0311x over previous
import functools

import jax
import jax.numpy as jnp
import numpy as np
from jax import lax
from jax.experimental import pallas as pl
from jax.experimental.pallas import tpu as pltpu
from jax.experimental.pallas import tpu_sc as plsc

_BATCH = 16384
_EMBED_DIM = 64
_OUT_FEATURES = 64
_NUM_CLASSES = 1000000
_PACK = 524288
_P2_ROWS = _PACK // 2
_ROW_W = 128
_PROJ_BLK = 16384

_NC = 2
_NS = 16
_NW = _NC * _NS
_B_PER_W = _BATCH // _NW


def _tc_project(table_t, dense_kernel):
    he_const = 1.0 / np.sqrt(dense_kernel.shape[0])
    n_steps = _PACK // _PROJ_BLK
    last_blk = (_NUM_CLASSES - 1) // _PROJ_BLK

    def proj_body(xlo_ref, xhi_ref, w_ref, o_ref):
        wb = (w_ref[...] * he_const).astype(jnp.bfloat16)
        ys = []
        for x_ref in (xlo_ref, xhi_ref):
            xb = x_ref[...].astype(jnp.bfloat16)
            y = jax.lax.dot_general(
                xb, wb, (((0,), (0,)), ((), ())),
                preferred_element_type=jnp.float32,
            )
            ys.append(y.astype(jnp.bfloat16))
        packed = jnp.concatenate(ys, axis=1)
        o_ref[...] = pltpu.bitcast(packed, jnp.float32)

    qb = n_steps
    return pl.pallas_call(
        proj_body,
        grid=(n_steps,),
        in_specs=[
            pl.BlockSpec((_EMBED_DIM, _PROJ_BLK), lambda i: (0, i)),
            pl.BlockSpec(
                (_EMBED_DIM, _PROJ_BLK),
                lambda i: (0, jnp.minimum(i + qb, last_blk)),
            ),
            pl.BlockSpec((_EMBED_DIM, _OUT_FEATURES), lambda i: (0, 0)),
        ],
        out_specs=pl.BlockSpec((_PROJ_BLK // 2, _ROW_W), lambda i: (i, 0)),
        out_shape=jax.ShapeDtypeStruct((_P2_ROWS, _ROW_W), jnp.float32),
        compiler_params=pltpu.CompilerParams(
            dimension_semantics=("parallel",),
            fuse_transposed_lhs_in_matmul=True,
        ),
    )(table_t, table_t, dense_kernel)


def _sc_gather(p2, idx):
    mesh = plsc.VectorSubcoreMesh(core_axis_name="c", subcore_axis_name="s")

    @functools.partial(
        pl.kernel,
        mesh=mesh,
        out_type=jax.ShapeDtypeStruct((_BATCH, _ROW_W), jnp.float32),
        scratch_types=[
            pltpu.VMEM((_B_PER_W,), jnp.int32),
            pltpu.VMEM((_B_PER_W, _ROW_W), jnp.float32),
            pltpu.SemaphoreType.DMA,
        ],
    )
    def gather_kernel(table_hbm, idx_hbm, out_hbm, idx_v, rows_v, sem):
        wid = lax.axis_index("s") * _NC + lax.axis_index("c")
        base = wid * _B_PER_W
        pltpu.sync_copy(idx_hbm.at[pl.ds(base, _B_PER_W)], idx_v)
        pltpu.async_copy(table_hbm.at[idx_v], rows_v, sem).wait()
        pltpu.sync_copy(rows_v, out_hbm.at[pl.ds(base, _B_PER_W)])

    return gather_kernel(p2, idx)


def _tc_epilogue(g, sel, dense_bias):
    bias_col = dense_bias.reshape(_OUT_FEATURES, 1)
    blk = 8192

    def epi_body(g_ref, s_ref, b_ref, o_ref):
        gi = pltpu.bitcast(g_ref[...].T, jnp.int32)
        s = s_ref[...]
        lo_bits = gi << 16
        hi_bits = gi & jnp.int32(-65536)
        picked = jnp.where((s & 1) == 0, lo_bits, hi_bits)
        yf = pltpu.bitcast(picked, jnp.float32)
        d = _OUT_FEATURES
        y = jnp.where((s & 2) == 0, yf[:d, :], yf[d:, :])
        o_ref[...] = y + b_ref[...]

    return pl.pallas_call(
        epi_body,
        grid=(_BATCH // blk,),
        in_specs=[
            pl.BlockSpec((blk, _ROW_W), lambda i: (i, 0)),
            pl.BlockSpec((1, blk), lambda i: (0, i)),
            pl.BlockSpec((_OUT_FEATURES, 1), lambda i: (0, 0)),
        ],
        out_specs=pl.BlockSpec((_OUT_FEATURES, blk), lambda i: (0, i)),
        out_shape=jax.ShapeDtypeStruct((_OUT_FEATURES, _BATCH), jnp.float32),
    )(g, sel, bias_col)


def kernel(label, embed_table, dense_kernel, dense_bias):
    table_t = embed_table.T
    p2 = _tc_project(table_t, dense_kernel)
    lo = label % _PACK
    idx = lo // 2
    sel = ((lo % 2) + 2 * (label // _PACK)).astype(jnp.int32).reshape(1, _BATCH)
    g = _sc_gather(p2, idx)
    y_t = _tc_epilogue(g, sel, dense_bias)
    return y_t.T.reshape(-1, 8, 8, 1)


# speedup vs baseline: 3.6334x; 1.0058x over previous
import functools

import jax
import jax.numpy as jnp
import numpy as np
from jax import lax
from jax.experimental import pallas as pl
from jax.experimental.pallas import tpu as pltpu
from jax.experimental.pallas import tpu_sc as plsc

_BATCH = 16384
_EMBED_DIM = 64
_OUT_FEATURES = 64
_NUM_CLASSES = 1000000
_PACK = 524288
_P2_ROWS = _PACK // 2
_ROW_W = 128
_PROJ_BLK = 16384

_NC = 2
_NS = 16
_NW = _NC * _NS
_B_PER_W = _BATCH // _NW


def _tc_project(table_t, dense_kernel):
    he_const = 1.0 / np.sqrt(dense_kernel.shape[0])
    n_steps = _PACK // _PROJ_BLK
    last_blk = (_NUM_CLASSES - 1) // _PROJ_BLK

    def proj_body(xlo_ref, xhi_ref, w_ref, o_ref):
        wb = (w_ref[...] * he_const).astype(jnp.bfloat16)
        d = _OUT_FEATURES
        for k, x_ref in enumerate((xlo_ref, xhi_ref)):
            xb = x_ref[...].astype(jnp.bfloat16)
            y = jax.lax.dot_general(
                xb, wb, (((0,), (0,)), ((), ())),
                preferred_element_type=jnp.float32,
            )
            o_ref[:, k * d:(k + 1) * d] = pltpu.bitcast(
                y.astype(jnp.bfloat16), jnp.float32
            )

    qb = n_steps
    return pl.pallas_call(
        proj_body,
        grid=(n_steps,),
        in_specs=[
            pl.BlockSpec((_EMBED_DIM, _PROJ_BLK), lambda i: (0, i)),
            pl.BlockSpec(
                (_EMBED_DIM, _PROJ_BLK),
                lambda i: (0, jnp.minimum(i + qb, last_blk)),
            ),
            pl.BlockSpec((_EMBED_DIM, _OUT_FEATURES), lambda i: (0, 0)),
        ],
        out_specs=pl.BlockSpec((_PROJ_BLK // 2, _ROW_W), lambda i: (i, 0)),
        out_shape=jax.ShapeDtypeStruct((_P2_ROWS, _ROW_W), jnp.float32),
        compiler_params=pltpu.CompilerParams(
            dimension_semantics=("parallel",),
            fuse_transposed_lhs_in_matmul=True,
        ),
    )(table_t, table_t, dense_kernel)


def _sc_gather(p2, idx):
    mesh = plsc.VectorSubcoreMesh(core_axis_name="c", subcore_axis_name="s")

    @functools.partial(
        pl.kernel,
        mesh=mesh,
        out_type=jax.ShapeDtypeStruct((_BATCH, _ROW_W), jnp.float32),
        scratch_types=[
            pltpu.VMEM((_B_PER_W,), jnp.int32),
            pltpu.VMEM((_B_PER_W, _ROW_W), jnp.float32),
            pltpu.SemaphoreType.DMA,
        ],
    )
    def gather_kernel(table_hbm, idx_hbm, out_hbm, idx_v, rows_v, sem):
        wid = lax.axis_index("s") * _NC + lax.axis_index("c")
        base = wid * _B_PER_W
        pltpu.sync_copy(idx_hbm.at[pl.ds(base, _B_PER_W)], idx_v)
        pltpu.async_copy(table_hbm.at[idx_v], rows_v, sem).wait()
        pltpu.sync_copy(rows_v, out_hbm.at[pl.ds(base, _B_PER_W)])

    return gather_kernel(p2, idx)


def _tc_epilogue(g, sel, dense_bias):
    bias_col = dense_bias.reshape(_OUT_FEATURES, 1)
    blk = 8192

    def epi_body(g_ref, s_ref, b_ref, o_ref):
        gi = pltpu.bitcast(g_ref[...].T, jnp.int32)
        s = s_ref[...]
        lo_bits = gi << 16
        hi_bits = gi & jnp.int32(-65536)
        picked = jnp.where((s & 1) == 0, lo_bits, hi_bits)
        yf = pltpu.bitcast(picked, jnp.float32)
        d = _OUT_FEATURES
        y = jnp.where((s & 2) == 0, yf[:d, :], yf[d:, :])
        o_ref[...] = y + b_ref[...]

    return pl.pallas_call(
        epi_body,
        grid=(_BATCH // blk,),
        in_specs=[
            pl.BlockSpec((blk, _ROW_W), lambda i: (i, 0)),
            pl.BlockSpec((1, blk), lambda i: (0, i)),
            pl.BlockSpec((_OUT_FEATURES, 1), lambda i: (0, 0)),
        ],
        out_specs=pl.BlockSpec((_OUT_FEATURES, blk), lambda i: (0, i)),
        out_shape=jax.ShapeDtypeStruct((_OUT_FEATURES, _BATCH), jnp.float32),
    )(g, sel, bias_col)


def kernel(label, embed_table, dense_kernel, dense_bias):
    table_t = embed_table.T
    p2 = _tc_project(table_t, dense_kernel)
    lo = label % _PACK
    idx = lo // 2
    sel = ((lo % 2) + 2 * (label // _PACK)).astype(jnp.int32).reshape(1, _BATCH)
    g = _sc_gather(p2, idx)
    y_t = _tc_epilogue(g, sel, dense_bias)
    return y_t.T.reshape(-1, 8, 8, 1)
